# Initial kernel scaffold; baseline (speedup 1.0000x reference)
import jax
import jax.numpy as jnp
from jax.experimental import pallas as pl


def kernel(x, pe_weight):
    raise NotImplementedError("write your pallas kernel here")


# speedup vs baseline: 1.6020x; 1.6020x over previous
import functools

import jax
import jax.numpy as jnp
from jax import lax
from jax.experimental import pallas as pl
from jax.experimental.pallas import tpu as pltpu
from jax.experimental.pallas import tpu_sc as plsc

_ROWS = 8192
_D = 1024
_NC = 2
_NS = 16
_NW = _NC * _NS
_RPW = _ROWS // _NW
_CHUNK = 32
_NCHUNK = _RPW // _CHUNK
_NBUF = 2

_mesh = plsc.VectorSubcoreMesh(core_axis_name="c", subcore_axis_name="s")


@functools.partial(
    pl.kernel,
    out_type=jax.ShapeDtypeStruct((_ROWS, _D), jnp.float32),
    mesh=_mesh,
    scratch_types=[
        pltpu.VMEM((_NBUF, _CHUNK, _D), jnp.float32),
        pltpu.SemaphoreType.DMA,
        pltpu.SemaphoreType.DMA,
        pltpu.SemaphoreType.DMA,
        pltpu.SemaphoreType.DMA,
    ],
)
def _pe_copy(table_hbm, out_hbm, buf, sin0, sin1, sout0, sout1):
    wid = lax.axis_index("s") * _NC + lax.axis_index("c")
    base = wid * _RPW
    sins = (sin0, sin1)
    souts = (sout0, sout1)
    in_copies = [None] * _NBUF
    out_copies = [None] * _NBUF

    in_copies[0] = pltpu.async_copy(
        table_hbm.at[pl.ds(base, _CHUNK)], buf.at[0], sins[0])
    for i in range(_NCHUNK):
        b = i % _NBUF
        nb = (i + 1) % _NBUF
        if i + 1 < _NCHUNK:
            if out_copies[nb] is not None:
                out_copies[nb].wait()
                out_copies[nb] = None
            in_copies[nb] = pltpu.async_copy(
                table_hbm.at[pl.ds(base + (i + 1) * _CHUNK, _CHUNK)],
                buf.at[nb], sins[nb])
        in_copies[b].wait()
        out_copies[b] = pltpu.async_copy(
            buf.at[b], out_hbm.at[pl.ds(base + i * _CHUNK, _CHUNK)], souts[b])
    for b in range(_NBUF):
        if out_copies[b] is not None:
            out_copies[b].wait()


def kernel(x, pe_weight):
    del x
    return _pe_copy(pe_weight)[None]
